# Initial kernel scaffold; baseline (speedup 1.0000x reference)
import jax
import jax.numpy as jnp
from jax.experimental import pallas as pl


def kernel(output, mask, ind, target, code_weights):
    raise NotImplementedError("write your pallas kernel here")


# speedup vs baseline: 1.0169x; 1.0169x over previous
import functools

import jax
import jax.numpy as jnp
from jax import lax
from jax.experimental import pallas as pl
from jax.experimental.pallas import tpu as pltpu
from jax.experimental.pallas import tpu_sc as plsc

WEIGHT = 1.0
LANES = 16


@functools.lru_cache(maxsize=None)
def _build(B, DIM, HW, OBJ_PAD):
    NCHUNK = OBJ_PAD // LANES
    IDX_ROW = 128
    DPER = IDX_ROW // LANES
    NROW = OBJ_PAD * DIM // IDX_ROW
    assert DIM * LANES == IDX_ROW

    mesh = plsc.VectorSubcoreMesh(
        core_axis_name="c", subcore_axis_name="s", num_cores=1,
        num_subcores=LANES)

    @functools.partial(
        pl.kernel,
        out_type=jax.ShapeDtypeStruct((LANES,), jnp.float32),
        mesh=mesh,
        scratch_types=dict(
            ind_v=pltpu.VMEM((OBJ_PAD,), jnp.int32),
            mask_v=pltpu.VMEM((OBJ_PAD,), jnp.float32),
            targ_v=pltpu.VMEM((DIM, OBJ_PAD), jnp.float32),
            idx_v=pltpu.VMEM((NROW, IDX_ROW), jnp.int32),
            gath_v=pltpu.VMEM((NROW, IDX_ROW), jnp.float32),
            part_v=pltpu.VMEM((DIM + 1, LANES), jnp.float32),
            shared_v=pltpu.VMEM_SHARED((LANES, DIM + 1, LANES), jnp.float32),
            red_v=pltpu.VMEM((LANES, DIM + 1, LANES), jnp.float32),
            cw_v=pltpu.VMEM((LANES,), jnp.float32),
            out_v=pltpu.VMEM((LANES,), jnp.float32),
            sem=pltpu.SemaphoreType.DMA,
        ),
    )
    def reg_loss(flat_hbm, ind_hbm, mask_hbm, targ_hbm, cw_hbm, out_hbm,
                 ind_v, mask_v, targ_v, idx_v, gath_v, part_v, shared_v,
                 red_v, cw_v, out_v, sem):
        b = lax.axis_index("s")
        pltpu.sync_copy(ind_hbm.at[b], ind_v)
        pltpu.sync_copy(mask_hbm.at[b], mask_v)
        pltpu.sync_copy(targ_hbm.at[b], targ_v)

        base = b * (DIM * HW)
        for j in range(NCHUNK):
            iv = ind_v[pl.ds(j * LANES, LANES)] + base
            for d in range(DIM):
                idx_v[j, pl.ds(d * LANES, LANES)] = iv + d * HW

        copies = [
            pltpu.async_copy(flat_hbm.at[idx_v.at[j]], gath_v.at[j], sem)
            for j in range(NROW)
        ]
        for c in copies:
            c.wait()
        plsc.subcore_barrier()
        pl.delay(1000)
        zoff = jnp.minimum(pl.semaphore_read(sem).astype(jnp.int32), 0)

        acc = [jnp.zeros((LANES,), jnp.float32) for _ in range(DIM)]
        msum = jnp.zeros((LANES,), jnp.float32)
        for j in range(NCHUNK):
            m = mask_v[pl.ds(j * LANES, LANES)]
            msum = msum + m
            for d in range(DIM):
                p = gath_v[j, pl.ds(d * LANES + zoff, LANES)]
                t = targ_v[d, pl.ds(j * LANES, LANES)]
                acc[d] = acc[d] + jnp.abs(p * m - t * m)
        for d in range(DIM):
            part_v[d, :] = acc[d]
        part_v[DIM, :] = msum

        pltpu.sync_copy(part_v, shared_v.at[b])
        plsc.subcore_barrier()

        @pl.when(b == 0)
        def _():
            pl.delay(300)
            pltpu.sync_copy(shared_v, red_v)
            pltpu.sync_copy(cw_hbm, cw_v)
            tot = [jnp.zeros((LANES,), jnp.float32) for _ in range(DIM + 1)]
            for w in range(LANES):
                for d in range(DIM + 1):
                    tot[d] = tot[d] + red_v[w, d, :]
            lanes = lax.iota(jnp.int32, LANES)

            dnums = lax.GatherDimensionNumbers(
                offset_dims=(), collapsed_slice_dims=(0,),
                start_index_map=(0,))

            def permute(x, idx):
                return lax.gather(
                    x, idx[:, None], dnums, slice_sizes=(1,),
                    mode=lax.GatherScatterMode.PROMISE_IN_BOUNDS)

            def lane_sum(x):
                for sh in (8, 4, 2, 1):
                    x = x + permute(x, lanes ^ sh)
                return x

            objv = lane_sum(tot[DIM]) + 0.0001
            res = jnp.zeros((LANES,), jnp.float32)
            for d in range(DIM):
                res = jnp.where(lanes == d, lane_sum(tot[d]), res)
            out_v[...] = res * cw_v[...] * (WEIGHT / objv)
            pltpu.sync_copy(out_v, out_hbm)

    return reg_loss


def kernel(output, mask, ind, target, code_weights):
    B, DIM, H, W = output.shape
    HW = H * W
    max_obj = ind.shape[1]
    obj_pad = -(-max_obj // LANES) * LANES
    if (obj_pad * DIM) % 128 != 0:
        obj_pad = -(-max_obj // 128) * 128

    flat = output.reshape(B * DIM * HW)
    ind_p = jnp.zeros((B, obj_pad), jnp.int32).at[:, :max_obj].set(
        ind.astype(jnp.int32))
    mask_p = jnp.zeros((B, obj_pad), jnp.float32).at[:, :max_obj].set(
        mask.astype(jnp.float32))
    targ_p = jnp.zeros((B, DIM, obj_pad), jnp.float32).at[:, :, :max_obj].set(
        target.astype(jnp.float32).transpose(0, 2, 1))
    cw = jnp.zeros((LANES,), jnp.float32).at[:DIM].set(
        code_weights.astype(jnp.float32))
    return _build(B, DIM, HW, obj_pad)(flat, ind_p, mask_p, targ_p, cw)[:DIM]


# speedup vs baseline: 1.0230x; 1.0060x over previous
import functools

import jax
import jax.numpy as jnp
from jax import lax
from jax.experimental import pallas as pl
from jax.experimental.pallas import tpu as pltpu
from jax.experimental.pallas import tpu_sc as plsc

WEIGHT = 1.0
LANES = 16


@functools.lru_cache(maxsize=None)
def _build(B, DIM, HW, OBJ_PAD):
    NCHUNK = OBJ_PAD // LANES
    IDX_ROW = 128
    DPER = IDX_ROW // LANES
    NROW = OBJ_PAD * DIM // IDX_ROW
    assert DIM * LANES == IDX_ROW

    mesh = plsc.VectorSubcoreMesh(
        core_axis_name="c", subcore_axis_name="s", num_cores=1,
        num_subcores=LANES)

    @functools.partial(
        pl.kernel,
        out_type=jax.ShapeDtypeStruct((LANES,), jnp.float32),
        mesh=mesh,
        scratch_types=dict(
            ind_v=pltpu.VMEM((OBJ_PAD,), jnp.int32),
            mask_v=pltpu.VMEM((OBJ_PAD,), jnp.float32),
            targ_v=pltpu.VMEM((DIM * OBJ_PAD,), jnp.float32),
            idx_v=pltpu.VMEM((NROW, IDX_ROW), jnp.int32),
            gath_v=pltpu.VMEM((NROW, IDX_ROW), jnp.float32),
            part_v=pltpu.VMEM((DIM + 1, LANES), jnp.float32),
            shared_v=pltpu.VMEM_SHARED((LANES, DIM + 1, LANES), jnp.float32),
            red_v=pltpu.VMEM((LANES, DIM + 1, LANES), jnp.float32),
            cw_v=pltpu.VMEM((LANES,), jnp.float32),
            out_v=pltpu.VMEM((LANES,), jnp.float32),
            sem=pltpu.SemaphoreType.DMA,
        ),
    )
    def reg_loss(flat_hbm, ind_hbm, mask_hbm, targ_hbm, cw_hbm, out_hbm,
                 ind_v, mask_v, targ_v, idx_v, gath_v, part_v, shared_v,
                 red_v, cw_v, out_v, sem):
        b = lax.axis_index("s")
        pltpu.sync_copy(ind_hbm.at[pl.ds(b * OBJ_PAD, OBJ_PAD)], ind_v)
        pltpu.sync_copy(mask_hbm.at[pl.ds(b * OBJ_PAD, OBJ_PAD)], mask_v)
        pltpu.sync_copy(
            targ_hbm.at[pl.ds(b * DIM * OBJ_PAD, DIM * OBJ_PAD)], targ_v)

        base = b * (DIM * HW)
        for j in range(NCHUNK):
            iv = ind_v[pl.ds(j * LANES, LANES)] + base
            for d in range(DIM):
                idx_v[j, pl.ds(d * LANES, LANES)] = iv + d * HW

        copies = [
            pltpu.async_copy(flat_hbm.at[idx_v.at[j]], gath_v.at[j], sem)
            for j in range(NROW)
        ]
        for c in copies:
            c.wait()
        plsc.subcore_barrier()
        pl.delay(1000)
        zoff = jnp.minimum(pl.semaphore_read(sem).astype(jnp.int32), 0)

        acc = [jnp.zeros((LANES,), jnp.float32) for _ in range(DIM)]
        msum = jnp.zeros((LANES,), jnp.float32)
        for j in range(NCHUNK):
            m = mask_v[pl.ds(j * LANES, LANES)]
            msum = msum + m
            for d in range(DIM):
                p = gath_v[j, pl.ds(d * LANES + zoff, LANES)]
                t = targ_v[pl.ds(d * OBJ_PAD + j * LANES, LANES)]
                acc[d] = acc[d] + jnp.abs(p * m - t * m)
        for d in range(DIM):
            part_v[d, :] = acc[d]
        part_v[DIM, :] = msum

        pltpu.sync_copy(part_v, shared_v.at[b])
        plsc.subcore_barrier()

        @pl.when(b == 0)
        def _():
            pl.delay(300)
            pltpu.sync_copy(shared_v, red_v)
            pltpu.sync_copy(cw_hbm, cw_v)
            tot = [jnp.zeros((LANES,), jnp.float32) for _ in range(DIM + 1)]
            for w in range(LANES):
                for d in range(DIM + 1):
                    tot[d] = tot[d] + red_v[w, d, :]
            lanes = lax.iota(jnp.int32, LANES)

            dnums = lax.GatherDimensionNumbers(
                offset_dims=(), collapsed_slice_dims=(0,),
                start_index_map=(0,))

            def permute(x, idx):
                return lax.gather(
                    x, idx[:, None], dnums, slice_sizes=(1,),
                    mode=lax.GatherScatterMode.PROMISE_IN_BOUNDS)

            def lane_sum(x):
                for sh in (8, 4, 2, 1):
                    x = x + permute(x, lanes ^ sh)
                return x

            objv = lane_sum(tot[DIM]) + 0.0001
            res = jnp.zeros((LANES,), jnp.float32)
            for d in range(DIM):
                res = jnp.where(lanes == d, lane_sum(tot[d]), res)
            out_v[...] = res * cw_v[...] * (WEIGHT / objv)
            pltpu.sync_copy(out_v, out_hbm)

    return reg_loss


def kernel(output, mask, ind, target, code_weights):
    B, DIM, H, W = output.shape
    HW = H * W
    max_obj = ind.shape[1]
    obj_pad = -(-max_obj // LANES) * LANES
    if (obj_pad * DIM) % 128 != 0:
        obj_pad = -(-max_obj // 128) * 128

    flat = output.reshape(B * DIM * HW)
    ind_p = jnp.zeros((B, obj_pad), jnp.int32).at[:, :max_obj].set(
        ind.astype(jnp.int32)).reshape(-1)
    mask_p = jnp.zeros((B, obj_pad), jnp.float32).at[:, :max_obj].set(
        mask.astype(jnp.float32)).reshape(-1)
    targ_p = jnp.zeros((B, DIM, obj_pad), jnp.float32).at[:, :, :max_obj].set(
        target.astype(jnp.float32).transpose(0, 2, 1)).reshape(-1)
    cw = jnp.zeros((LANES,), jnp.float32).at[:DIM].set(
        code_weights.astype(jnp.float32))
    return _build(B, DIM, HW, obj_pad)(flat, ind_p, mask_p, targ_p, cw)[:DIM]


# speedup vs baseline: 1.0255x; 1.0024x over previous
import functools

import jax
import jax.numpy as jnp
from jax import lax
from jax.experimental import pallas as pl
from jax.experimental.pallas import tpu as pltpu
from jax.experimental.pallas import tpu_sc as plsc

WEIGHT = 1.0
LANES = 16


@functools.lru_cache(maxsize=None)
def _build(B, DIM, HW, OBJ_PAD):
    NCHUNK = OBJ_PAD // LANES
    IDX_ROW = 128
    NROW = OBJ_PAD * DIM // IDX_ROW
    assert DIM * LANES == IDX_ROW

    mesh = plsc.VectorSubcoreMesh(
        core_axis_name="c", subcore_axis_name="s", num_cores=1,
        num_subcores=LANES)

    @functools.partial(
        pl.kernel,
        out_type=jax.ShapeDtypeStruct((LANES,), jnp.float32),
        mesh=mesh,
        scratch_types=dict(
            ind_v=pltpu.VMEM((OBJ_PAD,), jnp.int32),
            mask_v=pltpu.VMEM((OBJ_PAD,), jnp.float32),
            targ_v=pltpu.VMEM((DIM * OBJ_PAD,), jnp.float32),
            idx_v=pltpu.VMEM((NROW, IDX_ROW), jnp.int32),
            gath_v=pltpu.VMEM((NROW, IDX_ROW), jnp.float32),
            part_v=pltpu.VMEM((DIM + 1, LANES), jnp.float32),
            shared_v=pltpu.VMEM_SHARED((LANES, DIM + 1, LANES), jnp.float32),
            red_v=pltpu.VMEM((LANES, DIM + 1, LANES), jnp.float32),
            cw_v=pltpu.VMEM((LANES,), jnp.float32),
            out_v=pltpu.VMEM((LANES,), jnp.float32),
            sem=pltpu.SemaphoreType.DMA,
        ),
    )
    def reg_loss(flat_hbm, ind_hbm, mask_hbm, targ_hbm, cw_hbm, out_hbm,
                 ind_v, mask_v, targ_v, idx_v, gath_v, part_v, shared_v,
                 red_v, cw_v, out_v, sem):
        b = lax.axis_index("s")
        pltpu.sync_copy(ind_hbm.at[pl.ds(b * OBJ_PAD, OBJ_PAD)], ind_v)
        pltpu.sync_copy(mask_hbm.at[pl.ds(b * OBJ_PAD, OBJ_PAD)], mask_v)
        pltpu.sync_copy(
            targ_hbm.at[pl.ds(b * DIM * OBJ_PAD, DIM * OBJ_PAD)], targ_v)

        base = b * (DIM * HW)
        for j in range(NCHUNK):
            iv = ind_v[pl.ds(j * LANES, LANES)] + base
            for d in range(DIM):
                idx_v[j, pl.ds(d * LANES, LANES)] = iv + d * HW

        copies = [
            pltpu.async_copy(flat_hbm.at[idx_v.at[j]], gath_v.at[j], sem)
            for j in range(NROW)
        ]
        for c in copies:
            c.wait()
        plsc.subcore_barrier()
        pl.delay(1000)
        zoff = jnp.minimum(pl.semaphore_read(sem).astype(jnp.int32), 0)

        acc = [jnp.zeros((LANES,), jnp.float32) for _ in range(DIM)]
        msum = jnp.zeros((LANES,), jnp.float32)
        for j in range(NCHUNK):
            m = mask_v[pl.ds(j * LANES, LANES)]
            msum = msum + m
            for d in range(DIM):
                p = gath_v[j, pl.ds(d * LANES + zoff, LANES)]
                t = targ_v[pl.ds(d * OBJ_PAD + j * LANES, LANES)]
                acc[d] = acc[d] + jnp.abs(p * m - t * m)
        for d in range(DIM):
            part_v[d, :] = acc[d]
        part_v[DIM, :] = msum

        pltpu.sync_copy(part_v, shared_v.at[b])
        plsc.subcore_barrier()

        @pl.when(b == 0)
        def _():
            pl.delay(300)
            pltpu.sync_copy(shared_v, red_v)
            pltpu.sync_copy(cw_hbm, cw_v)
            tot = [jnp.zeros((LANES,), jnp.float32) for _ in range(DIM + 1)]
            for w in range(LANES):
                for d in range(DIM + 1):
                    tot[d] = tot[d] + red_v[w, d, :]
            lanes = lax.iota(jnp.int32, LANES)

            dnums = lax.GatherDimensionNumbers(
                offset_dims=(), collapsed_slice_dims=(0,),
                start_index_map=(0,))

            def permute(x, idx):
                return lax.gather(
                    x, idx[:, None], dnums, slice_sizes=(1,),
                    mode=lax.GatherScatterMode.PROMISE_IN_BOUNDS)

            def lane_sum(x):
                for sh in (8, 4, 2, 1):
                    x = x + permute(x, lanes ^ sh)
                return x

            objv = lane_sum(tot[DIM]) + 0.0001
            res = jnp.zeros((LANES,), jnp.float32)
            for d in range(DIM):
                res = jnp.where(lanes == d, lane_sum(tot[d]), res)
            out_v[...] = res * cw_v[...] * (WEIGHT / objv)
            pltpu.sync_copy(out_v, out_hbm)

    return reg_loss


def kernel(output, mask, ind, target, code_weights):
    B, DIM, H, W = output.shape
    HW = H * W
    max_obj = ind.shape[1]
    obj_pad = -(-max_obj // LANES) * LANES
    if (obj_pad * DIM) % 128 != 0:
        obj_pad = -(-max_obj // 128) * 128

    flat = output.reshape(B * DIM * HW)
    ind_p = jnp.zeros((B, obj_pad), jnp.int32).at[:, :max_obj].set(
        ind.astype(jnp.int32)).reshape(-1)
    mask_p = jnp.zeros((B, obj_pad), jnp.float32).at[:, :max_obj].set(
        mask.astype(jnp.float32)).reshape(-1)
    targ_p = jnp.zeros((B, DIM, obj_pad), jnp.float32).at[:, :, :max_obj].set(
        target.astype(jnp.float32).transpose(0, 2, 1)).reshape(-1)
    cw = jnp.zeros((LANES,), jnp.float32).at[:DIM].set(
        code_weights.astype(jnp.float32))
    return _build(B, DIM, HW, obj_pad)(flat, ind_p, mask_p, targ_p, cw)[:DIM]
